# Initial kernel scaffold; baseline (speedup 1.0000x reference)
import jax
import jax.numpy as jnp
from jax.experimental import pallas as pl


def kernel(x, edge_index, edge_weight, W1, b1, W2, b2):
    raise NotImplementedError("write your pallas kernel here")


# speedup vs baseline: 9.6798x; 9.6798x over previous
import functools

import jax
import jax.numpy as jnp
from jax import lax
from jax.experimental import pallas as pl
from jax.experimental.pallas import tpu as pltpu
from jax.experimental.pallas import tpu_sc as plsc

N = 10000
E = 320000
D_IN = 128
D_H = 128
D_LAT = 64

NC = 2
NS = 16
NW = NC * NS
CE = 80
KJD = (E // NW) // CE
KJS = (E // NS) // CE
NST = 2
KB = KJS // NST
HN = N // NC
AROWS = HN + 8
OSC = 5
ORW = HN // OSC
ZR = 40

_mesh = plsc.VectorSubcoreMesh(core_axis_name="c", subcore_axis_name="s")


@functools.partial(
    pl.kernel,
    out_type=jax.ShapeDtypeStruct((NC, N), jnp.float32),
    mesh=_mesh,
    scratch_types=[
        pltpu.VMEM((KJD, CE), jnp.int32),
        pltpu.VMEM((KJD, CE), jnp.float32),
        pltpu.VMEM((2000,), jnp.float32),
        pltpu.VMEM_SHARED((N,), jnp.float32),
    ],
)
def _deg_kernel(dst_hbm, ew_hbm, out_hbm, dst_v, ew_v, zero_v, deg_sh):
    cid = lax.axis_index("c")
    sid = lax.axis_index("s")
    wid = sid * NC + cid

    @pl.when(sid == 0)
    def _zero():
        def zfill(i, c):
            zero_v[pl.ds(i * 16, 16)] = jnp.zeros((16,), jnp.float32)
            return c
        lax.fori_loop(0, 125, zfill, 0)

        def zcopy(i, c):
            pltpu.sync_copy(zero_v, deg_sh.at[pl.ds(i * 2000, 2000)])
            return c
        lax.fori_loop(0, N // 2000, zcopy, 0)

    plsc.subcore_barrier()

    pltpu.sync_copy(dst_hbm.at[wid], dst_v)
    pltpu.sync_copy(ew_hbm.at[wid], ew_v)

    def inner(j, c):
        pltpu.sync_copy(ew_v.at[j], deg_sh.at[dst_v.at[j]], add=True)
        return c
    lax.fori_loop(0, KJD, inner, 0)

    plsc.subcore_barrier()

    @pl.when(sid == 0)
    def _out():
        pltpu.sync_copy(deg_sh, out_hbm.at[cid])


def _make_scatter(nv):
    @functools.partial(
        pl.kernel,
        out_type=jax.ShapeDtypeStruct((NC, HN, D_H), jnp.float32),
        mesh=_mesh,
        scratch_types=[
            pltpu.VMEM((KB, CE), jnp.int32),
            pltpu.VMEM((KB, CE), jnp.int32),
            pltpu.VMEM((KB, CE), jnp.float32),
            pltpu.VMEM((CE, D_H), jnp.float32),
            pltpu.VMEM((ZR, D_H), jnp.float32),
            pltpu.VMEM_SHARED((AROWS, D_H), jnp.float32),
            pltpu.SemaphoreType.DMA,
        ],
    )
    def _scatter(g_hbm, src_hbm, dst_hbm, ew_hbm, out_hbm,
                 src_v, dst_v, ew_v, rows_v, zero_v, acc_sh, sem):
        cid = lax.axis_index("c")
        sid = lax.axis_index("s")
        row0 = cid * HN

        @pl.when(sid < OSC)
        def _zero():
            def zfill(i, c):
                for d in range(D_H // 16):
                    zero_v[i, pl.ds(d * 16, 16)] = jnp.zeros((16,), jnp.float32)
                return c
            lax.fori_loop(0, ZR, zfill, 0)

            def zcopy(i, c):
                pltpu.sync_copy(zero_v, acc_sh.at[pl.ds(sid * ORW + i * ZR, ZR)])
                return c
            lax.fori_loop(0, ORW // ZR, zcopy, 0)

        plsc.subcore_barrier()

        def stage(st, c0):
            pltpu.sync_copy(src_hbm.at[sid, st], src_v)
            pltpu.sync_copy(dst_hbm.at[sid, st], dst_v)
            pltpu.sync_copy(ew_hbm.at[sid, st], ew_v)

            def remap(j, c):
                for g in range(CE // 16):
                    d16 = dst_v[j, pl.ds(g * 16, 16)] - row0
                    ok = (d16 >= 0) & (d16 < HN)
                    dst_v[j, pl.ds(g * 16, 16)] = jnp.where(ok, d16, HN)
                return c
            lax.fori_loop(0, KB, remap, 0)

            def inner(j, c):
                pltpu.async_copy(g_hbm.at[src_v.at[j]], rows_v, sem).wait()

                def scale(g, c2):
                    wvec = ew_v[j, pl.ds(g * 16, 16)]
                    base_e = g * 16
                    for l in range(16):
                        w = wvec[l]
                        for d in range(nv):
                            rows_v[base_e + l, pl.ds(d * 16, 16)] = (
                                rows_v[base_e + l, pl.ds(d * 16, 16)] * w)
                    return c2
                lax.fori_loop(0, CE // 16, scale, 0)

                pltpu.sync_copy(rows_v, acc_sh.at[dst_v.at[j]], add=True)
                return c
            lax.fori_loop(0, KB, inner, 0)
            return c0
        lax.fori_loop(0, NST, stage, 0)

        plsc.subcore_barrier()

        @pl.when(sid < OSC)
        def _out():
            pltpu.sync_copy(acc_sh.at[pl.ds(sid * ORW, ORW)],
                            out_hbm.at[cid, pl.ds(sid * ORW, ORW)])

    return _scatter


_scatter_h = _make_scatter(D_H // 16)
_scatter_lat = _make_scatter(D_LAT // 16)


def _tc1_body(degp_ref, x_ref, w1_ref, g1_ref, dinv_ref):
    deg = degp_ref[0] + degp_ref[1] + 1.0
    dinv = lax.rsqrt(deg)
    dinv_ref[...] = dinv
    g1_ref[...] = jnp.dot(x_ref[...], w1_ref[...],
                          preferred_element_type=jnp.float32) * dinv[:, None]


_tc1 = pl.pallas_call(
    _tc1_body,
    out_shape=(jax.ShapeDtypeStruct((N, D_H), jnp.float32),
               jax.ShapeDtypeStruct((N,), jnp.float32)),
)


def _tc2_body(acc_ref, g1_ref, dinv_ref, b1_ref, w2_ref, g2_ref):
    dinv = dinv_ref[...]
    msg = jnp.concatenate([acc_ref[0], acc_ref[1]], axis=0)
    h1 = jnp.maximum((msg + g1_ref[...]) * dinv[:, None] + b1_ref[...], 0.0)
    g2 = jnp.dot(h1, w2_ref[...],
                 preferred_element_type=jnp.float32) * dinv[:, None]
    g2_ref[...] = jnp.concatenate(
        [g2, jnp.zeros((N, D_H - D_LAT), jnp.float32)], axis=1)


_tc2 = pl.pallas_call(
    _tc2_body,
    out_shape=jax.ShapeDtypeStruct((N, D_H), jnp.float32),
)


def _tc3_body(acc_ref, g2_ref, dinv_ref, b2_ref, out_ref):
    dinv = dinv_ref[...]
    msg = jnp.concatenate([acc_ref[0], acc_ref[1]], axis=0)[:, :D_LAT]
    out_ref[...] = jnp.maximum((msg + g2_ref[:, :D_LAT]) * dinv[:, None]
                               + b2_ref[...], 0.0)


_tc3 = pl.pallas_call(
    _tc3_body,
    out_shape=jax.ShapeDtypeStruct((N, D_LAT), jnp.float32),
)


def kernel(x, edge_index, edge_weight, W1, b1, W2, b2):
    src_d = edge_index[0].reshape(NW, KJD, CE)
    dst_d = edge_index[1].reshape(NW, KJD, CE)
    ew_d = edge_weight.reshape(NW, KJD, CE)
    src_s = edge_index[0].reshape(NS, NST, KB, CE)
    dst_s = edge_index[1].reshape(NS, NST, KB, CE)
    ew_s = edge_weight.reshape(NS, NST, KB, CE)

    degp = _deg_kernel(dst_d, ew_d)
    g1, dinv = _tc1(degp, x, W1)

    acc1 = _scatter_h(g1, src_s, dst_s, ew_s)
    g2p = _tc2(acc1, g1, dinv, b1, W2)

    acc2 = _scatter_lat(g2p, src_s, dst_s, ew_s)
    return _tc3(acc2, g2p, dinv, b2)


# speedup vs baseline: 15.2439x; 1.5748x over previous
import functools

import jax
import jax.numpy as jnp
from jax import lax
from jax.experimental import pallas as pl
from jax.experimental.pallas import tpu as pltpu
from jax.experimental.pallas import tpu_sc as plsc

N = 10000
E = 320000
D_IN = 128
D_H = 128
D_LAT = 64

NC = 2
NS = 16
NW = NC * NS
CE = 80
KJD = (E // NW) // CE
KJS = (E // NS) // CE
NST = 2
KB = KJS // NST
HN = N // NC
AROWS = HN + 8
OSC = 5
ORW = HN // OSC
ZR = 40

_mesh = plsc.VectorSubcoreMesh(core_axis_name="c", subcore_axis_name="s")


@functools.partial(
    pl.kernel,
    out_type=jax.ShapeDtypeStruct((NC, N), jnp.float32),
    mesh=_mesh,
    scratch_types=[
        pltpu.VMEM((KJD, CE), jnp.int32),
        pltpu.VMEM((KJD, CE), jnp.float32),
        pltpu.VMEM((2000,), jnp.float32),
        pltpu.VMEM_SHARED((N,), jnp.float32),
    ],
)
def _deg_kernel(dst_hbm, ew_hbm, out_hbm, dst_v, ew_v, zero_v, deg_sh):
    cid = lax.axis_index("c")
    sid = lax.axis_index("s")
    wid = sid * NC + cid

    @pl.when(sid == 0)
    def _zero():
        def zfill(i, c):
            zero_v[pl.ds(i * 16, 16)] = jnp.zeros((16,), jnp.float32)
            return c
        lax.fori_loop(0, 125, zfill, 0)

        def zcopy(i, c):
            pltpu.sync_copy(zero_v, deg_sh.at[pl.ds(i * 2000, 2000)])
            return c
        lax.fori_loop(0, N // 2000, zcopy, 0)

    plsc.subcore_barrier()

    pltpu.sync_copy(dst_hbm.at[wid], dst_v)
    pltpu.sync_copy(ew_hbm.at[wid], ew_v)

    def inner(j, c):
        pltpu.sync_copy(ew_v.at[j], deg_sh.at[dst_v.at[j]], add=True)
        return c
    lax.fori_loop(0, KJD, inner, 0)

    plsc.subcore_barrier()

    @pl.when(sid == 0)
    def _out():
        pltpu.sync_copy(deg_sh, out_hbm.at[cid])


def _make_scatter(nv):
    @functools.partial(
        pl.kernel,
        out_type=jax.ShapeDtypeStruct((NC, HN, D_H), jnp.float32),
        mesh=_mesh,
        scratch_types=[
            pltpu.VMEM((KB, CE), jnp.int32),
            pltpu.VMEM((KB, CE), jnp.int32),
            pltpu.VMEM((KB, CE), jnp.float32),
            pltpu.VMEM((CE, D_H), jnp.float32),
            pltpu.VMEM((CE, D_H), jnp.float32),
            pltpu.VMEM((ZR, D_H), jnp.float32),
            pltpu.VMEM_SHARED((AROWS, D_H), jnp.float32),
            pltpu.SemaphoreType.DMA,
            pltpu.SemaphoreType.DMA,
        ],
    )
    def _scatter(g_hbm, src_hbm, dst_hbm, ew_hbm, out_hbm,
                 src_v, dst_v, ew_v, rows_a, rows_b, zero_v, acc_sh,
                 sem_a, sem_b):
        cid = lax.axis_index("c")
        sid = lax.axis_index("s")
        row0 = cid * HN

        @pl.when(sid < OSC)
        def _zero():
            def zfill(i, c):
                for d in range(D_H // 16):
                    zero_v[i, pl.ds(d * 16, 16)] = jnp.zeros((16,), jnp.float32)
                return c
            lax.fori_loop(0, ZR, zfill, 0)

            def zcopy(i, c):
                pltpu.sync_copy(zero_v, acc_sh.at[pl.ds(sid * ORW + i * ZR, ZR)])
                return c
            lax.fori_loop(0, ORW // ZR, zcopy, 0)

        plsc.subcore_barrier()

        def stage(st, c0):
            pltpu.sync_copy(src_hbm.at[sid, st], src_v)
            pltpu.sync_copy(dst_hbm.at[sid, st], dst_v)
            pltpu.sync_copy(ew_hbm.at[sid, st], ew_v)

            def remap(j, c):
                for g in range(CE // 16):
                    d16 = dst_v[j, pl.ds(g * 16, 16)] - row0
                    ok = (d16 >= 0) & (d16 < HN)
                    dst_v[j, pl.ds(g * 16, 16)] = jnp.where(ok, d16, HN)
                return c
            lax.fori_loop(0, KB, remap, 0)

            def process(j, rows_v):
                def scale(g, c2):
                    wvec = ew_v[j, pl.ds(g * 16, 16)]
                    base_e = g * 16
                    for l in range(16):
                        w = wvec[l]
                        for d in range(nv):
                            rows_v[base_e + l, pl.ds(d * 16, 16)] = (
                                rows_v[base_e + l, pl.ds(d * 16, 16)] * w)
                    return c2
                lax.fori_loop(0, CE // 16, scale, 0)
                pltpu.sync_copy(rows_v, acc_sh.at[dst_v.at[j]], add=True)

            pltpu.async_copy(g_hbm.at[src_v.at[0]], rows_a, sem_a)

            def pair(p, c):
                j0 = p * 2
                pltpu.async_copy(g_hbm.at[src_v.at[j0 + 1]], rows_b, sem_b)
                pltpu.make_async_copy(g_hbm.at[src_v.at[j0]], rows_a,
                                      sem_a).wait()
                process(j0, rows_a)
                pltpu.async_copy(g_hbm.at[src_v.at[j0 + 2]], rows_a, sem_a)
                pltpu.make_async_copy(g_hbm.at[src_v.at[j0 + 1]], rows_b,
                                      sem_b).wait()
                process(j0 + 1, rows_b)
                return c
            lax.fori_loop(0, (KB - 1) // 2, pair, 0)

            pltpu.make_async_copy(g_hbm.at[src_v.at[KB - 1]], rows_a,
                                  sem_a).wait()
            process(KB - 1, rows_a)
            return c0
        lax.fori_loop(0, NST, stage, 0)

        plsc.subcore_barrier()

        @pl.when(sid < OSC)
        def _out():
            pltpu.sync_copy(acc_sh.at[pl.ds(sid * ORW, ORW)],
                            out_hbm.at[cid, pl.ds(sid * ORW, ORW)])

    return _scatter


_scatter_h = _make_scatter(D_H // 16)
_scatter_lat = _make_scatter(D_LAT // 16)


def _tc1_body(degp_ref, x_ref, w1_ref, g1_ref, dinv_ref):
    deg = degp_ref[0] + degp_ref[1] + 1.0
    dinv = lax.rsqrt(deg)
    dinv_ref[...] = dinv
    g1_ref[...] = jnp.dot(x_ref[...], w1_ref[...],
                          preferred_element_type=jnp.float32) * dinv[:, None]


_tc1 = pl.pallas_call(
    _tc1_body,
    out_shape=(jax.ShapeDtypeStruct((N, D_H), jnp.float32),
               jax.ShapeDtypeStruct((N,), jnp.float32)),
)


def _tc2_body(acc_ref, g1_ref, dinv_ref, b1_ref, w2_ref, g2_ref):
    dinv = dinv_ref[...]
    msg = jnp.concatenate([acc_ref[0], acc_ref[1]], axis=0)
    h1 = jnp.maximum((msg + g1_ref[...]) * dinv[:, None] + b1_ref[...], 0.0)
    g2 = jnp.dot(h1, w2_ref[...],
                 preferred_element_type=jnp.float32) * dinv[:, None]
    g2_ref[...] = jnp.concatenate(
        [g2, jnp.zeros((N, D_H - D_LAT), jnp.float32)], axis=1)


_tc2 = pl.pallas_call(
    _tc2_body,
    out_shape=jax.ShapeDtypeStruct((N, D_H), jnp.float32),
)


def _tc3_body(acc_ref, g2_ref, dinv_ref, b2_ref, out_ref):
    dinv = dinv_ref[...]
    msg = jnp.concatenate([acc_ref[0], acc_ref[1]], axis=0)[:, :D_LAT]
    out_ref[...] = jnp.maximum((msg + g2_ref[:, :D_LAT]) * dinv[:, None]
                               + b2_ref[...], 0.0)


_tc3 = pl.pallas_call(
    _tc3_body,
    out_shape=jax.ShapeDtypeStruct((N, D_LAT), jnp.float32),
)


def kernel(x, edge_index, edge_weight, W1, b1, W2, b2):
    src_d = edge_index[0].reshape(NW, KJD, CE)
    dst_d = edge_index[1].reshape(NW, KJD, CE)
    ew_d = edge_weight.reshape(NW, KJD, CE)
    src_s = edge_index[0].reshape(NS, NST, KB, CE)
    dst_s = edge_index[1].reshape(NS, NST, KB, CE)
    ew_s = edge_weight.reshape(NS, NST, KB, CE)

    degp = _deg_kernel(dst_d, ew_d)
    g1, dinv = _tc1(degp, x, W1)

    acc1 = _scatter_h(g1, src_s, dst_s, ew_s)
    g2p = _tc2(acc1, g1, dinv, b1, W2)

    acc2 = _scatter_lat(g2p, src_s, dst_s, ew_s)
    return _tc3(acc2, g2p, dinv, b2)


# speedup vs baseline: 25.3766x; 1.6647x over previous
import functools

import jax
import jax.numpy as jnp
from jax import lax
from jax.experimental import pallas as pl
from jax.experimental.pallas import tpu as pltpu
from jax.experimental.pallas import tpu_sc as plsc

N = 10000
E = 320000
D_IN = 128
D_H = 128
D_LAT = 64

NC = 2
NS = 16
NW = NC * NS
CE = 80
KJD = (E // NW) // CE
KJS = (E // NS) // CE
NST = 2
KB = KJS // NST
HN = N // NC
AROWS = HN + 8
OSC = 5
ORW = HN // OSC
ZR = 40

NSR = 5
KBR = KJD // NSR
OSCF = 10
ORWF = N // OSCF
ZRF = 8

_mesh = plsc.VectorSubcoreMesh(core_axis_name="c", subcore_axis_name="s")


@functools.partial(
    pl.kernel,
    out_type=jax.ShapeDtypeStruct((NC, N), jnp.float32),
    mesh=_mesh,
    scratch_types=[
        pltpu.VMEM((KJD, CE), jnp.int32),
        pltpu.VMEM((KJD, CE), jnp.float32),
        pltpu.VMEM((2000,), jnp.float32),
        pltpu.VMEM_SHARED((N,), jnp.float32),
    ],
)
def _deg_kernel(dst_hbm, ew_hbm, out_hbm, dst_v, ew_v, zero_v, deg_sh):
    cid = lax.axis_index("c")
    sid = lax.axis_index("s")
    wid = sid * NC + cid

    @pl.when(sid == 0)
    def _zero():
        def zfill(i, c):
            zero_v[pl.ds(i * 16, 16)] = jnp.zeros((16,), jnp.float32)
            return c
        lax.fori_loop(0, 125, zfill, 0)

        def zcopy(i, c):
            pltpu.sync_copy(zero_v, deg_sh.at[pl.ds(i * 2000, 2000)])
            return c
        lax.fori_loop(0, N // 2000, zcopy, 0)

    plsc.subcore_barrier()

    pltpu.sync_copy(dst_hbm.at[wid], dst_v)
    pltpu.sync_copy(ew_hbm.at[wid], ew_v)

    def inner(j, c):
        pltpu.sync_copy(ew_v.at[j], deg_sh.at[dst_v.at[j]], add=True)
        return c
    lax.fori_loop(0, KJD, inner, 0)

    plsc.subcore_barrier()

    @pl.when(sid == 0)
    def _out():
        pltpu.sync_copy(deg_sh, out_hbm.at[cid])


def _make_scatter(nv):
    @functools.partial(
        pl.kernel,
        out_type=jax.ShapeDtypeStruct((NC, N, D_H), jnp.float32),
        mesh=_mesh,
        scratch_types=[
            pltpu.VMEM((KBR, CE), jnp.int32),
            pltpu.VMEM((KBR, CE), jnp.int32),
            pltpu.VMEM((KBR, CE), jnp.float32),
            pltpu.VMEM((CE, D_H), jnp.float32),
            pltpu.VMEM((CE, D_H), jnp.float32),
            pltpu.VMEM((ZRF, D_H), jnp.float32),
            pltpu.VMEM_SHARED((N, D_H), jnp.float32),
            pltpu.SemaphoreType.DMA,
            pltpu.SemaphoreType.DMA,
        ],
    )
    def _scatter(g_hbm, src_hbm, dst_hbm, ew_hbm, out_hbm,
                 src_v, dst_v, ew_v, rows_a, rows_b, zero_v, acc_sh,
                 sem_a, sem_b):
        cid = lax.axis_index("c")
        sid = lax.axis_index("s")
        wid = sid * NC + cid

        @pl.when(sid < OSCF)
        def _zero():
            def zfill(i, c):
                for d in range(D_H // 16):
                    zero_v[i, pl.ds(d * 16, 16)] = jnp.zeros((16,), jnp.float32)
                return c
            lax.fori_loop(0, ZRF, zfill, 0)

            def zcopy(i, c):
                pltpu.sync_copy(zero_v,
                                acc_sh.at[pl.ds(sid * ORWF + i * ZRF, ZRF)])
                return c
            lax.fori_loop(0, ORWF // ZRF, zcopy, 0)

        plsc.subcore_barrier()

        def process(j, rows_v):
            def scale(g, c2):
                wvec = ew_v[j, pl.ds(g * 16, 16)]
                base_e = g * 16
                for l in range(16):
                    w = wvec[l]
                    for d in range(nv):
                        rows_v[base_e + l, pl.ds(d * 16, 16)] = (
                            rows_v[base_e + l, pl.ds(d * 16, 16)] * w)
                return c2
            lax.fori_loop(0, CE // 16, scale, 0)
            pltpu.sync_copy(rows_v, acc_sh.at[dst_v.at[j]], add=True)

        def rnd(st, c0):
            pltpu.sync_copy(src_hbm.at[wid, st], src_v)
            pltpu.sync_copy(dst_hbm.at[wid, st], dst_v)
            pltpu.sync_copy(ew_hbm.at[wid, st], ew_v)

            pltpu.async_copy(g_hbm.at[src_v.at[0]], rows_a, sem_a)

            def pair(p, c):
                j0 = p * 2
                pltpu.async_copy(g_hbm.at[src_v.at[j0 + 1]], rows_b, sem_b)
                pltpu.make_async_copy(g_hbm.at[src_v.at[j0]], rows_a,
                                      sem_a).wait()
                process(j0, rows_a)
                pltpu.async_copy(g_hbm.at[src_v.at[j0 + 2]], rows_a, sem_a)
                pltpu.make_async_copy(g_hbm.at[src_v.at[j0 + 1]], rows_b,
                                      sem_b).wait()
                process(j0 + 1, rows_b)
                return c
            lax.fori_loop(0, (KBR - 1) // 2, pair, 0)

            pltpu.make_async_copy(g_hbm.at[src_v.at[KBR - 1]], rows_a,
                                  sem_a).wait()
            process(KBR - 1, rows_a)
            return c0
        lax.fori_loop(0, NSR, rnd, 0)

        plsc.subcore_barrier()

        @pl.when(sid < OSCF)
        def _out():
            pltpu.sync_copy(acc_sh.at[pl.ds(sid * ORWF, ORWF)],
                            out_hbm.at[cid, pl.ds(sid * ORWF, ORWF)])

    return _scatter


_scatter_h = _make_scatter(D_H // 16)
_scatter_lat = _make_scatter(D_LAT // 16)


def _tc1_body(degp_ref, x_ref, w1_ref, g1_ref, dinv_ref):
    deg = degp_ref[0] + degp_ref[1] + 1.0
    dinv = lax.rsqrt(deg)
    dinv_ref[...] = dinv
    g1_ref[...] = jnp.dot(x_ref[...], w1_ref[...],
                          preferred_element_type=jnp.float32) * dinv[:, None]


_tc1 = pl.pallas_call(
    _tc1_body,
    out_shape=(jax.ShapeDtypeStruct((N, D_H), jnp.float32),
               jax.ShapeDtypeStruct((N,), jnp.float32)),
)


def _tc2_body(acc_ref, g1_ref, dinv_ref, b1_ref, w2_ref, g2_ref):
    dinv = dinv_ref[...]
    msg = acc_ref[0] + acc_ref[1]
    h1 = jnp.maximum((msg + g1_ref[...]) * dinv[:, None] + b1_ref[...], 0.0)
    g2 = jnp.dot(h1, w2_ref[...],
                 preferred_element_type=jnp.float32) * dinv[:, None]
    g2_ref[...] = jnp.concatenate(
        [g2, jnp.zeros((N, D_H - D_LAT), jnp.float32)], axis=1)


_tc2 = pl.pallas_call(
    _tc2_body,
    out_shape=jax.ShapeDtypeStruct((N, D_H), jnp.float32),
)


def _tc3_body(acc_ref, g2_ref, dinv_ref, b2_ref, out_ref):
    dinv = dinv_ref[...]
    msg = (acc_ref[0] + acc_ref[1])[:, :D_LAT]
    out_ref[...] = jnp.maximum((msg + g2_ref[:, :D_LAT]) * dinv[:, None]
                               + b2_ref[...], 0.0)


_tc3 = pl.pallas_call(
    _tc3_body,
    out_shape=jax.ShapeDtypeStruct((N, D_LAT), jnp.float32),
)


def kernel(x, edge_index, edge_weight, W1, b1, W2, b2):
    src_d = edge_index[0].reshape(NW, KJD, CE)
    dst_d = edge_index[1].reshape(NW, KJD, CE)
    ew_d = edge_weight.reshape(NW, KJD, CE)
    src_s = edge_index[0].reshape(NW, NSR, KBR, CE)
    dst_s = edge_index[1].reshape(NW, NSR, KBR, CE)
    ew_s = edge_weight.reshape(NW, NSR, KBR, CE)

    degp = _deg_kernel(dst_d, ew_d)
    g1, dinv = _tc1(degp, x, W1)

    acc1 = _scatter_h(g1, src_s, dst_s, ew_s)
    g2p = _tc2(acc1, g1, dinv, b1, W2)

    acc2 = _scatter_lat(g2p, src_s, dst_s, ew_s)
    return _tc3(acc2, g2p, dinv, b2)
